# Initial kernel scaffold; baseline (speedup 1.0000x reference)
import jax
import jax.numpy as jnp
from jax.experimental import pallas as pl


def kernel(xyz, points, W0, b0, gamma0, beta0, W1, b1, gamma1, beta1, W2, b2, gamma2, beta2):
    raise NotImplementedError("write your pallas kernel here")


# speedup vs baseline: 11.4125x; 11.4125x over previous
import functools

import jax
import jax.numpy as jnp
from jax import lax
from jax.experimental import pallas as pl
from jax.experimental.pallas import tpu as pltpu
from jax.experimental.pallas import tpu_sc as plsc

B, N, D = 4, 8192, 32
NPOINT, NSAMPLE = 2048, 32
EPS = 1e-5

TS = 128
ROWS = B * NPOINT * NSAMPLE
PROWS = ROWS // 4


NCHK = 32
CW = N // NCHK
PRE = 8


def _full_extract(d):
    iota = lax.broadcasted_iota(jnp.int32, (TS, N), 1)
    big = jnp.float32(jnp.inf)
    cols = []
    for _ in range(NSAMPLE):
        gmin = jnp.min(d, axis=1, keepdims=True)
        am = jnp.min(jnp.where(d == gmin, iota, N), axis=1, keepdims=True)
        cols.append(am)
        d = jnp.where(iota == am, big, d)
    return jnp.concatenate(cols, axis=1)


def _knn_body(q_ref, p_ref, qn_ref, pn_ref, w0_ref, b0_ref, idx_ref, pw_ref):
    b = pl.program_id(0)
    q = q_ref[0]
    p = p_ref[0]
    qn = qn_ref[0]
    pn = pn_ref[0]
    pw_ref[0] = lax.dot_general(p, w0_ref[...], (((1,), (1,)), ((), ())),
                                preferred_element_type=jnp.float32
                                ) + b0_ref[...]
    qp = lax.dot_general(q, p, (((1,), (1,)), ((), ())),
                         preferred_element_type=jnp.float32)
    d = -2.0 * qp + qn + pn
    big = jnp.float32(jnp.inf)

    d3 = d.reshape(TS, NCHK, CW)
    iota3 = (lax.broadcasted_iota(jnp.int32, (TS, NCHK, CW), 1) * CW
             + lax.broadcasted_iota(jnp.int32, (TS, NCHK, CW), 2))
    dwork = d3
    cvals, cidxs = [], []
    for _ in range(PRE):
        cmin = jnp.min(dwork, axis=2, keepdims=True)
        am = jnp.min(jnp.where(dwork == cmin, iota3, N), axis=2,
                     keepdims=True)
        cvals.append(cmin)
        cidxs.append(am)
        dwork = jnp.where(iota3 == am, big, dwork)
    cv = jnp.concatenate(cvals, axis=2).reshape(TS, NCHK * PRE)
    ci = jnp.concatenate(cidxs, axis=2).reshape(TS, NCHK * PRE)

    cols = []
    for _ in range(NSAMPLE):
        g = jnp.min(cv, axis=1, keepdims=True)
        am = jnp.min(jnp.where(cv == g, ci, N), axis=1, keepdims=True)
        cols.append(am)
        cv = jnp.where(ci == am, big, cv)
    idx_cand = jnp.concatenate(cols, axis=1)

    selchunk = idx_cand // CW
    occ = jnp.sum((selchunk[:, :, None]
                   == lax.broadcasted_iota(jnp.int32, (1, 1, NCHK), 2)
                   ).astype(jnp.int32), axis=1)
    bad = jnp.max(occ) >= PRE

    @pl.when(jnp.logical_not(bad))
    def _():
        idx_ref[0] = idx_cand + b * N

    @pl.when(bad)
    def _():
        idx_ref[0] = _full_extract(d) + b * N


def _knn(new_points, points, W0, b0):
    qn = jnp.sum(new_points ** 2, axis=-1)[:, :, None]
    pn = jnp.sum(points ** 2, axis=-1)[:, None, :]
    grid = (B, NPOINT // TS)
    return pl.pallas_call(
        _knn_body,
        grid=grid,
        in_specs=[
            pl.BlockSpec((1, TS, D), lambda b, s: (b, s, 0)),
            pl.BlockSpec((1, N, D), lambda b, s: (b, 0, 0)),
            pl.BlockSpec((1, TS, 1), lambda b, s: (b, s, 0)),
            pl.BlockSpec((1, 1, N), lambda b, s: (b, 0, 0)),
            pl.BlockSpec((D, D), lambda b, s: (0, 0)),
            pl.BlockSpec((1, D), lambda b, s: (0, 0)),
        ],
        out_specs=[
            pl.BlockSpec((1, TS, NSAMPLE), lambda b, s: (b, s, 0)),
            pl.BlockSpec((1, N, D), lambda b, s: (b, 0, 0)),
        ],
        out_shape=[
            jax.ShapeDtypeStruct((B, NPOINT, NSAMPLE), jnp.int32),
            jax.ShapeDtypeStruct((B, N, D), jnp.float32),
        ],
    )(new_points, points, qn, pn, W0, b0.reshape(1, D))


_IDX_ROWS = ROWS // 128
_RPW = _IDX_ROWS // 32
_CHUNK = 1024
_PCHUNK = _CHUNK // 4


def _gather_body(table_hbm, idx_hbm, out_hbm, st_hbm,
                 idx_v, rows_v, pack_v, st_v, sem):
    wid = lax.axis_index("s") * 2 + lax.axis_index("c")
    pltpu.sync_copy(idx_hbm.at[pl.ds(wid * _RPW, _RPW)], idx_v)
    zero = jnp.zeros((16,), jnp.float32)
    acc = (zero, zero, zero, zero)
    for r in range(8):
        waits = []
        for j in range(8):
            waits.append(pltpu.async_copy(
                table_hbm.at[idx_v.at[r * 8 + j]],
                rows_v.at[pl.ds(j * 128, 128)], sem))
        for w in waits:
            w.wait()

        def body(i, acc):
            s0, s1, q0, q1 = acc
            pr = i
            for c in range(4):
                lo = rows_v[4 * pr + c, pl.ds(0, 16)]
                hi = rows_v[4 * pr + c, pl.ds(16, 16)]
                s0 += lo
                s1 += hi
                q0 += lo * lo
                q1 += hi * hi
                pack_v[pr, pl.ds(c * 32, 16)] = lo
                pack_v[pr, pl.ds(c * 32 + 16, 16)] = hi
            return (s0, s1, q0, q1)

        acc = lax.fori_loop(0, _PCHUNK, body, acc)
        pltpu.sync_copy(
            pack_v, out_hbm.at[pl.ds(wid * (8192 // 4) + r * _PCHUNK,
                                     _PCHUNK)])
    s0, s1, q0, q1 = acc
    st_v[0, pl.ds(0, 16)] = s0
    st_v[0, pl.ds(16, 16)] = s1
    st_v[1, pl.ds(0, 16)] = q0
    st_v[1, pl.ds(16, 16)] = q1
    pltpu.sync_copy(st_v, st_hbm.at[wid])


def _gather_stats(table, idx):
    k = functools.partial(
        pl.kernel,
        out_type=[
            jax.ShapeDtypeStruct((PROWS, 128), jnp.float32),
            jax.ShapeDtypeStruct((32, 2, 32), jnp.float32),
        ],
        compiler_params=pltpu.CompilerParams(use_tc_tiling_on_sc=False),
        mesh=plsc.VectorSubcoreMesh(core_axis_name="c", subcore_axis_name="s"),
        scratch_types=[
            pltpu.VMEM((_RPW, 128), jnp.int32),
            pltpu.VMEM((_CHUNK, D), jnp.float32),
            pltpu.VMEM((_PCHUNK, 128), jnp.float32),
            pltpu.VMEM((2, 32), jnp.float32),
            pltpu.SemaphoreType.DMA,
        ],
    )(_gather_body)
    return k(table, idx)


MLP_R = 2048


def _mlp_mid_body(x_ref, ac_ref, w_ref, b_ref, y_ref, st_ref):
    x = x_ref[...]
    a = ac_ref[0:1, :]
    c = ac_ref[1:2, :]
    h = jnp.maximum(x * a + c, 0.0)
    y = lax.dot_general(h, w_ref[...], (((1,), (0,)), ((), ())),
                        preferred_element_type=jnp.float32) + b_ref[...]
    y_ref[...] = y
    s = jnp.sum(y, axis=0, keepdims=True)
    s2 = jnp.sum(y * y, axis=0, keepdims=True)
    st = jnp.concatenate([s, s2], axis=0)

    @pl.when(pl.program_id(0) == 0)
    def _():
        st_ref[...] = jnp.zeros_like(st_ref)

    st_ref[...] += st


def _mlp_mid(x, ac, wd, bd):
    oc4 = wd.shape[1]
    grid = (PROWS // MLP_R,)
    return pl.pallas_call(
        _mlp_mid_body,
        grid=grid,
        in_specs=[
            pl.BlockSpec((MLP_R, 128), lambda i: (i, 0)),
            pl.BlockSpec((2, 128), lambda i: (0, 0)),
            pl.BlockSpec((128, oc4), lambda i: (0, 0)),
            pl.BlockSpec((1, oc4), lambda i: (0, 0)),
        ],
        out_specs=[
            pl.BlockSpec((MLP_R, oc4), lambda i: (i, 0)),
            pl.BlockSpec((2, oc4), lambda i: (0, 0)),
        ],
        out_shape=[
            jax.ShapeDtypeStruct((PROWS, oc4), jnp.float32),
            jax.ShapeDtypeStruct((2, oc4), jnp.float32),
        ],
    )(x, ac, wd, bd.reshape(1, oc4))


def _mlp_last_body(x_ref, ac_ref, o_ref):
    x = x_ref[...]
    a = ac_ref[0:1, :]
    c = ac_ref[1:2, :]
    h = jnp.maximum(x * a + c, 0.0)
    r = h.shape[0]
    h8 = h.reshape(r // 8, 8, 256)
    m8 = jnp.max(h8, axis=1)
    m4 = m8.reshape(r // 8, 4, 64)
    o_ref[...] = jnp.max(m4, axis=1)


def _mlp_last(x, ac):
    grid = (PROWS // MLP_R,)
    return pl.pallas_call(
        _mlp_last_body,
        grid=grid,
        in_specs=[
            pl.BlockSpec((MLP_R, 256), lambda i: (i, 0)),
            pl.BlockSpec((2, 256), lambda i: (0, 0)),
        ],
        out_specs=pl.BlockSpec((MLP_R // 8, 64), lambda i: (i, 0)),
        out_shape=jax.ShapeDtypeStruct((ROWS // NSAMPLE, 64), jnp.float32),
    )(x, ac)


def _affine(st, gamma, beta):
    mean = st[0] / ROWS
    var = st[1] / ROWS - mean * mean
    a = gamma * lax.rsqrt(var + EPS)
    c = beta - mean * a
    return a, c


def kernel(xyz, points, W0, b0, gamma0, beta0, W1, b1, gamma1, beta1,
           W2, b2, gamma2, beta2):
    idx_perm = jax.random.permutation(jax.random.key(42), N)[:NPOINT]
    new_xyz = jnp.take(xyz, idx_perm, axis=1)
    new_points = jnp.take(points, idx_perm, axis=1)

    flat_idx, pw1 = _knn(new_points, points, W0, b0)
    y1p, st1p = _gather_stats(pw1.reshape(B * N, D),
                              flat_idx.reshape(_IDX_ROWS, 128))

    st1 = jnp.sum(st1p, axis=0)
    a1, c1 = _affine(st1, gamma0, beta0)
    ac1 = jnp.stack([jnp.tile(a1, 4), jnp.tile(c1, 4)], axis=0)
    wd1 = jnp.kron(jnp.eye(4, dtype=jnp.float32), W1.T)

    y2p, st2p = _mlp_mid(y1p, ac1, wd1, jnp.tile(b1, 4))
    st2 = st2p.reshape(2, 4, 32).sum(axis=1)
    a2, c2 = _affine(st2, gamma1, beta1)
    ac2 = jnp.stack([jnp.tile(a2, 4), jnp.tile(c2, 4)], axis=0)
    wd2 = jnp.kron(jnp.eye(4, dtype=jnp.float32), W2.T)

    y3p, st3p = _mlp_mid(y2p, ac2, wd2, jnp.tile(b2, 4))
    st3 = st3p.reshape(2, 4, 64).sum(axis=1)
    a3, c3 = _affine(st3, gamma2, beta2)
    ac3 = jnp.stack([jnp.tile(a3, 4), jnp.tile(c3, 4)], axis=0)

    out = _mlp_last(y3p, ac3)
    return (new_xyz, out.reshape(B, NPOINT, W2.shape[0]))


# speedup vs baseline: 12.4018x; 1.0867x over previous
import functools

import jax
import jax.numpy as jnp
from jax import lax
from jax.experimental import pallas as pl
from jax.experimental.pallas import tpu as pltpu
from jax.experimental.pallas import tpu_sc as plsc

B, N, D = 4, 8192, 32
NPOINT, NSAMPLE = 2048, 32
EPS = 1e-5

TS = 128
ROWS = B * NPOINT * NSAMPLE
PROWS = ROWS // 4


NCHK = 32
CW = N // NCHK
PRE = 8


def _full_extract(d):
    iota = lax.broadcasted_iota(jnp.int32, (TS, N), 1)
    big = jnp.float32(jnp.inf)
    cols = []
    for _ in range(NSAMPLE):
        gmin = jnp.min(d, axis=1, keepdims=True)
        am = jnp.min(jnp.where(d == gmin, iota, N), axis=1, keepdims=True)
        cols.append(am)
        d = jnp.where(iota == am, big, d)
    return jnp.concatenate(cols, axis=1)


def _knn_body(q_ref, p_ref, qn_ref, pn_ref, w0_ref, b0_ref, idx_ref, pw_ref):
    b = pl.program_id(0)
    q = q_ref[0]
    p = p_ref[0]
    qn = qn_ref[0]
    pn = pn_ref[0]
    pw_ref[0] = lax.dot_general(p, w0_ref[...], (((1,), (1,)), ((), ())),
                                preferred_element_type=jnp.float32
                                ) + b0_ref[...]
    qp = lax.dot_general(q, p, (((1,), (1,)), ((), ())),
                         preferred_element_type=jnp.float32)
    d = -2.0 * qp + qn + pn
    big = jnp.float32(jnp.inf)

    d3 = d.reshape(TS, NCHK, CW)
    iota3 = (lax.broadcasted_iota(jnp.int32, (TS, NCHK, CW), 1) * CW
             + lax.broadcasted_iota(jnp.int32, (TS, NCHK, CW), 2))
    dwork = d3
    cvals, cidxs = [], []
    for _ in range(PRE):
        cmin = jnp.min(dwork, axis=2, keepdims=True)
        eq = dwork == cmin
        am = jnp.min(jnp.where(eq, iota3, N), axis=2, keepdims=True)
        cvals.append(cmin)
        cidxs.append(am)
        dwork = jnp.where(eq, big, dwork)
    cv = jnp.concatenate(cvals, axis=2).reshape(TS, NCHK * PRE)
    ci = jnp.concatenate(cidxs, axis=2).reshape(TS, NCHK * PRE)

    cols = []
    g = None
    for _ in range(NSAMPLE):
        g = jnp.min(cv, axis=1, keepdims=True)
        eq = cv == g
        am = jnp.min(jnp.where(eq, ci, N), axis=1, keepdims=True)
        cols.append(am)
        cv = jnp.where(eq, big, cv)
    idx_cand = jnp.concatenate(cols, axis=1)

    cnt = jnp.sum((d <= g).astype(jnp.int32), axis=1, keepdims=True)
    bad = jnp.max(jnp.abs(cnt - NSAMPLE)) > 0

    @pl.when(jnp.logical_not(bad))
    def _():
        idx_ref[0] = idx_cand + b * N

    @pl.when(bad)
    def _():
        idx_ref[0] = _full_extract(d) + b * N


def _knn(new_points, points, W0, b0):
    qn = jnp.sum(new_points ** 2, axis=-1)[:, :, None]
    pn = jnp.sum(points ** 2, axis=-1)[:, None, :]
    grid = (B, NPOINT // TS)
    return pl.pallas_call(
        _knn_body,
        grid=grid,
        in_specs=[
            pl.BlockSpec((1, TS, D), lambda b, s: (b, s, 0)),
            pl.BlockSpec((1, N, D), lambda b, s: (b, 0, 0)),
            pl.BlockSpec((1, TS, 1), lambda b, s: (b, s, 0)),
            pl.BlockSpec((1, 1, N), lambda b, s: (b, 0, 0)),
            pl.BlockSpec((D, D), lambda b, s: (0, 0)),
            pl.BlockSpec((1, D), lambda b, s: (0, 0)),
        ],
        out_specs=[
            pl.BlockSpec((1, TS, NSAMPLE), lambda b, s: (b, s, 0)),
            pl.BlockSpec((1, N, D), lambda b, s: (b, 0, 0)),
        ],
        out_shape=[
            jax.ShapeDtypeStruct((B, NPOINT, NSAMPLE), jnp.int32),
            jax.ShapeDtypeStruct((B, N, D), jnp.float32),
        ],
    )(new_points, points, qn, pn, W0, b0.reshape(1, D))


_IDX_ROWS = ROWS // 128
_RPW = _IDX_ROWS // 32
_CHUNK = 1024
_PCHUNK = _CHUNK // 4


def _gather_body(table_hbm, idx_hbm, out_hbm, st_hbm,
                 idx_v, rows_v, pack_v, st_v, sem):
    wid = lax.axis_index("s") * 2 + lax.axis_index("c")
    pltpu.sync_copy(idx_hbm.at[pl.ds(wid * _RPW, _RPW)], idx_v)
    zero = jnp.zeros((16,), jnp.float32)
    acc = (zero, zero, zero, zero)
    for r in range(8):
        waits = []
        for j in range(8):
            waits.append(pltpu.async_copy(
                table_hbm.at[idx_v.at[r * 8 + j]],
                rows_v.at[pl.ds(j * 128, 128)], sem))
        for w in waits:
            w.wait()

        def body(i, acc):
            s0, s1, q0, q1 = acc
            pr = i
            for c in range(4):
                lo = rows_v[4 * pr + c, pl.ds(0, 16)]
                hi = rows_v[4 * pr + c, pl.ds(16, 16)]
                s0 += lo
                s1 += hi
                q0 += lo * lo
                q1 += hi * hi
                pack_v[pr, pl.ds(c * 32, 16)] = lo
                pack_v[pr, pl.ds(c * 32 + 16, 16)] = hi
            return (s0, s1, q0, q1)

        acc = lax.fori_loop(0, _PCHUNK, body, acc)
        pltpu.sync_copy(
            pack_v, out_hbm.at[pl.ds(wid * (8192 // 4) + r * _PCHUNK,
                                     _PCHUNK)])
    s0, s1, q0, q1 = acc
    st_v[0, pl.ds(0, 16)] = s0
    st_v[0, pl.ds(16, 16)] = s1
    st_v[1, pl.ds(0, 16)] = q0
    st_v[1, pl.ds(16, 16)] = q1
    pltpu.sync_copy(st_v, st_hbm.at[wid])


def _gather_stats(table, idx):
    k = functools.partial(
        pl.kernel,
        out_type=[
            jax.ShapeDtypeStruct((PROWS, 128), jnp.float32),
            jax.ShapeDtypeStruct((32, 2, 32), jnp.float32),
        ],
        compiler_params=pltpu.CompilerParams(use_tc_tiling_on_sc=False),
        mesh=plsc.VectorSubcoreMesh(core_axis_name="c", subcore_axis_name="s"),
        scratch_types=[
            pltpu.VMEM((_RPW, 128), jnp.int32),
            pltpu.VMEM((_CHUNK, D), jnp.float32),
            pltpu.VMEM((_PCHUNK, 128), jnp.float32),
            pltpu.VMEM((2, 32), jnp.float32),
            pltpu.SemaphoreType.DMA,
        ],
    )(_gather_body)
    return k(table, idx)


MLP_R = 2048


def _layers(x, params):
    h = x
    for (ac, wd, bd) in params:
        h = jnp.maximum(h * ac[0:1, :] + ac[1:2, :], 0.0)
        h = lax.dot_general(h, wd, (((1,), (0,)), ((), ())),
                            preferred_element_type=jnp.float32) + bd
    return h


def _mlp_stats_body(nlayer):
    def body(*refs):
        x_ref = refs[0]
        params = [(refs[1 + 3 * i][...], refs[2 + 3 * i][...],
                   refs[3 + 3 * i][...]) for i in range(nlayer)]
        st_ref = refs[1 + 3 * nlayer]
        y = _layers(x_ref[...], params)
        s = jnp.sum(y, axis=0, keepdims=True)
        s2 = jnp.sum(y * y, axis=0, keepdims=True)
        st = jnp.concatenate([s, s2], axis=0)

        @pl.when(pl.program_id(0) == 0)
        def _():
            st_ref[...] = jnp.zeros_like(st_ref)

        st_ref[...] += st
    return body


def _mlp_stats(x, params):
    oc4 = params[-1][1].shape[1]
    grid = (PROWS // MLP_R,)
    ins = [x]
    in_specs = [pl.BlockSpec((MLP_R, 128), lambda i: (i, 0))]
    for (ac, wd, bd) in params:
        ins += [ac, wd, bd.reshape(1, -1)]
        in_specs += [
            pl.BlockSpec(ac.shape, lambda i: (0, 0)),
            pl.BlockSpec(wd.shape, lambda i: (0, 0)),
            pl.BlockSpec((1, bd.shape[0]), lambda i: (0, 0)),
        ]
    return pl.pallas_call(
        _mlp_stats_body(len(params)),
        grid=grid,
        in_specs=in_specs,
        out_specs=pl.BlockSpec((2, oc4), lambda i: (0, 0)),
        out_shape=jax.ShapeDtypeStruct((2, oc4), jnp.float32),
    )(*ins)


def _mlp_out_body(nlayer):
    def body(*refs):
        x_ref = refs[0]
        params = [(refs[1 + 3 * i][...], refs[2 + 3 * i][...],
                   refs[3 + 3 * i][...]) for i in range(nlayer)]
        ac3 = refs[1 + 3 * nlayer]
        o_ref = refs[2 + 3 * nlayer]
        y = _layers(x_ref[...], params)
        h = jnp.maximum(y * ac3[0:1, :] + ac3[1:2, :], 0.0)
        r = h.shape[0]
        m8 = jnp.max(h.reshape(r // 8, 8, 256), axis=1)
        o_ref[...] = jnp.max(m8.reshape(r // 8, 4, 64), axis=1)
    return body


def _mlp_out(x, params, ac3):
    grid = (PROWS // MLP_R,)
    ins = [x]
    in_specs = [pl.BlockSpec((MLP_R, 128), lambda i: (i, 0))]
    for (ac, wd, bd) in params:
        ins += [ac, wd, bd.reshape(1, -1)]
        in_specs += [
            pl.BlockSpec(ac.shape, lambda i: (0, 0)),
            pl.BlockSpec(wd.shape, lambda i: (0, 0)),
            pl.BlockSpec((1, bd.shape[0]), lambda i: (0, 0)),
        ]
    ins.append(ac3)
    in_specs.append(pl.BlockSpec((2, 256), lambda i: (0, 0)))
    return pl.pallas_call(
        _mlp_out_body(len(params)),
        grid=grid,
        in_specs=in_specs,
        out_specs=pl.BlockSpec((MLP_R // 8, 64), lambda i: (i, 0)),
        out_shape=jax.ShapeDtypeStruct((ROWS // NSAMPLE, 64), jnp.float32),
    )(*ins)


def _affine(st, gamma, beta):
    mean = st[0] / ROWS
    var = st[1] / ROWS - mean * mean
    a = gamma * lax.rsqrt(var + EPS)
    c = beta - mean * a
    return a, c


def kernel(xyz, points, W0, b0, gamma0, beta0, W1, b1, gamma1, beta1,
           W2, b2, gamma2, beta2):
    idx_perm = jax.random.permutation(jax.random.key(42), N)[:NPOINT]
    new_xyz = jnp.take(xyz, idx_perm, axis=1)
    new_points = jnp.take(points, idx_perm, axis=1)

    flat_idx, pw1 = _knn(new_points, points, W0, b0)
    y1p, st1p = _gather_stats(pw1.reshape(B * N, D),
                              flat_idx.reshape(_IDX_ROWS, 128))

    st1 = jnp.sum(st1p, axis=0)
    a1, c1 = _affine(st1, gamma0, beta0)
    ac1 = jnp.stack([jnp.tile(a1, 4), jnp.tile(c1, 4)], axis=0)
    wd1 = jnp.kron(jnp.eye(4, dtype=jnp.float32), W1.T)
    bd1 = jnp.tile(b1, 4)
    p1 = (ac1, wd1, bd1)

    st2 = _mlp_stats(y1p, [p1]).reshape(2, 4, 32).sum(axis=1)
    a2, c2 = _affine(st2, gamma1, beta1)
    ac2 = jnp.stack([jnp.tile(a2, 4), jnp.tile(c2, 4)], axis=0)
    wd2 = jnp.kron(jnp.eye(4, dtype=jnp.float32), W2.T)
    bd2 = jnp.tile(b2, 4)
    p2 = (ac2, wd2, bd2)

    st3 = _mlp_stats(y1p, [p1, p2]).reshape(2, 4, 64).sum(axis=1)
    a3, c3 = _affine(st3, gamma2, beta2)
    ac3 = jnp.stack([jnp.tile(a3, 4), jnp.tile(c3, 4)], axis=0)

    out = _mlp_out(y1p, [p1, p2], ac3)
    return (new_xyz, out.reshape(B, NPOINT, W2.shape[0]))
